# baseline (speedup 1.0000x reference)
import functools

import jax
import jax.numpy as jnp
from jax import lax
from jax.experimental import pallas as pl
from jax.experimental.pallas import tpu as pltpu
from jax.experimental.pallas import tpu_sc as plsc

VOCAB = 100000
D = 64
W = D // 2
LN_EPS = 1e-12

B, F, L = 1024, 26, 50
S = B * F

NC, NS = 2, 16
NW = NC * NS

ROWS_MAIN = 3200
ROWS_LAST = VOCAB - 31 * ROWS_MAIN
RC = 160

SEG_W = S // NW
CH = 16
NCHUNK = SEG_W // CH
TPG = 100
GPC = CH * L // TPG

_SC_PARAMS = pltpu.CompilerParams(
    use_tc_tiling_on_sc=False, needs_layout_passes=False)


def _ln_body(tab_hbm, out_hbm, in_v, out_v, isem, osem):
    wid = lax.axis_index("s") * NC + lax.axis_index("c")
    base = wid * ROWS_MAIN
    nch = jnp.where(wid < NW - 1, ROWS_MAIN // RC, ROWS_LAST // RC)

    iota = lax.iota(jnp.int32, 16)
    half = jnp.full((16,), 0.5, jnp.float32)
    three_half = jnp.full((16,), 1.5, jnp.float32)
    inv_d = jnp.full((16,), 1.0 / D, jnp.float32)
    eps = jnp.full((16,), LN_EPS, jnp.float32)
    zero = jnp.zeros((16,), jnp.float32)

    def issue_in(c, p):
        row0 = pl.multiple_of(base + c * RC, 8)
        return pltpu.async_copy(
            tab_hbm.at[pl.ds(row0, RC)], in_v.at[p, :, pl.ds(0, D)], isem[p])

    def wait_in(p):
        pltpu.make_async_copy(
            tab_hbm.at[pl.ds(0, RC)], in_v.at[p, :, pl.ds(0, D)],
            isem[p]).wait()

    def issue_out(c, p):
        row0 = pl.multiple_of(base + c * RC, 8)
        return pltpu.async_copy(
            out_v.at[p, :, pl.ds(0, W)], out_hbm.at[pl.ds(row0, RC)], osem[p])

    def wait_out(p):
        pltpu.make_async_copy(
            out_v.at[p, :, pl.ds(0, W)], out_hbm.at[pl.ds(0, RC)],
            osem[p]).wait()

    def ln_group(p, r0):
        rowv = iota + r0
        ss = [zero] * 8
        qq = [zero] * 8
        for d in range(D):
            dv = jnp.full((16,), d, jnp.int32)
            x = plsc.load_gather(in_v.at[p], [rowv, dv])
            ss[d % 8] = ss[d % 8] + x
            qq[d % 8] = x * x + qq[d % 8]
        while len(ss) > 1:
            ss = [a + b for a, b in zip(ss[::2], ss[1::2])]
            qq = [a + b for a, b in zip(qq[::2], qq[1::2])]
        mu = ss[0] * inv_d
        var = qq[0] * inv_d - mu * mu
        t = jnp.maximum(var, 0.0) + eps
        y = plsc.bitcast(
            jnp.full((16,), 0x5F3759DF, jnp.int32)
            - (plsc.bitcast(t, jnp.int32) >> 1), jnp.float32)
        ht = t * half
        for _ in range(2):
            y = y * (three_half - ht * y * y)
        c0 = mu * y

        for hb in range(2):
            ybuf = []
            for j in range(16):
                dv = jnp.full((16,), 32 * hb + j, jnp.int32)
                x = plsc.load_gather(in_v.at[p], [rowv, dv])
                ybuf.append(x * y - c0)
            for j in range(16):
                dv = jnp.full((16,), 32 * hb + 16 + j, jnp.int32)
                x = plsc.load_gather(in_v.at[p], [rowv, dv])
                yd = x * y - c0
                wrd = plsc.bitcast(
                    plsc.pack(ybuf[j], yd, format=plsc.PackFormat.INTERLEAVED),
                    jnp.int32)
                wv = jnp.full((16,), 16 * hb + j, jnp.int32)
                plsc.store_scatter(out_v.at[p], [rowv, wv], wrd)

    issue_in(0, 0)
    issue_in(1, 1)

    @pl.loop(0, NCHUNK_LN_MAX, step=2)
    def _chunk(c0):
        for par in (0, 1):
            c = c0 + par

            @pl.when(c < nch)
            def _():
                wait_in(par)

                @pl.when(c >= 2)
                def _():
                    wait_out(par)

                @pl.loop(0, RC // 16)
                def _grp(g):
                    ln_group(par, g * 16)

                issue_out(c, par)

                @pl.when(c + 2 < nch)
                def _():
                    issue_in(c + 2, par)

    wait_out(0)
    wait_out(1)


NCHUNK_LN_MAX = ROWS_MAIN // RC


def _normalize_table(table):
    mesh = plsc.VectorSubcoreMesh(core_axis_name="c", subcore_axis_name="s")
    f = functools.partial(
        pl.kernel,
        out_type=jax.ShapeDtypeStruct((VOCAB, W), jnp.int32),
        mesh=mesh,
        compiler_params=_SC_PARAMS,
        scratch_types=[
            pltpu.VMEM((2, RC, D + 1), jnp.float32),
            pltpu.VMEM((2, RC, W + 1), jnp.int32),
            [pltpu.SemaphoreType.DMA, pltpu.SemaphoreType.DMA],
            [pltpu.SemaphoreType.DMA, pltpu.SemaphoreType.DMA],
        ],
    )(_ln_body)
    return f(table)


def _seg_sum_body(ids_hbm, tabn_hbm, gamma_hbm, beta_hbm, out_hbm,
                  idx_v, rows_v, out_v, gb_v, isem, gsem, osem):
    wid = lax.axis_index("s") * NC + lax.axis_index("c")
    base_seg = wid * SEG_W
    base_row = wid * (SEG_W * L // TPG)

    pltpu.sync_copy(gamma_hbm, gb_v.at[pl.ds(0, D)])
    pltpu.sync_copy(beta_hbm, gb_v.at[pl.ds(D, D)])
    gvec = [gb_v[pl.ds(16 * k, 16)] for k in range(4)]
    bvec = [gb_v[pl.ds(D + 16 * k, 16)] * 50.0 for k in range(4)]

    def issue_idx(c, p):
        row0 = pl.multiple_of(base_row + c * GPC, 8)
        return pltpu.async_copy(
            ids_hbm.at[pl.ds(row0, GPC)], idx_v.at[p], isem[p])

    def wait_idx(p):
        pltpu.make_async_copy(
            ids_hbm.at[pl.ds(0, GPC)], idx_v.at[p], isem[p]).wait()

    def issue_gathers(p):
        for j in range(GPC):
            pltpu.async_copy(
                tabn_hbm.at[idx_v.at[p, j]],
                rows_v.at[p, pl.ds(j * TPG, TPG)],
                gsem[p])

    def wait_gathers(p):
        for j in range(GPC):
            pltpu.make_async_copy(
                tabn_hbm.at[idx_v.at[p, j]],
                rows_v.at[p, pl.ds(j * TPG, TPG)],
                gsem[p]).wait()

    def issue_out(c, p):
        seg0 = pl.multiple_of(base_seg + c * CH, 8)
        return pltpu.async_copy(
            out_v.at[p], out_hbm.at[pl.ds(seg0, CH)], osem[p])

    def wait_out(p):
        pltpu.make_async_copy(
            out_v.at[p], out_hbm.at[pl.ds(0, CH)], osem[p]).wait()

    def compute(p):
        @pl.loop(0, CH)
        def _seg(s):
            base = s * L
            acc = [jnp.zeros((16,), jnp.float32) for _ in range(4)]
            for t in range(L // 2):
                r0 = base + 2 * t
                for h in range(2):
                    a = plsc.bitcast(
                        rows_v[p, r0, pl.ds(h * 16, 16)], jnp.bfloat16)
                    b = plsc.bitcast(
                        rows_v[p, r0 + 1, pl.ds(h * 16, 16)], jnp.bfloat16)
                    pair = plsc.bitcast(a + b, jnp.int32)
                    lo = plsc.bitcast(pair << 16, jnp.float32)
                    hi = plsc.bitcast(pair & jnp.int32(-65536), jnp.float32)
                    acc[2 * h] = acc[2 * h] + lo
                    acc[2 * h + 1] = acc[2 * h + 1] + hi
            for k in range(4):
                out_v[p, s, pl.ds(16 * k, 16)] = acc[k] * gvec[k] + bvec[k]

    issue_idx(0, 0)
    wait_idx(0)
    issue_gathers(0)
    issue_idx(1, 1)

    @pl.loop(0, NCHUNK, step=2)
    def _chunk(c0):
        for par in (0, 1):
            c = c0 + par

            @pl.when(c + 1 < NCHUNK)
            def _():
                wait_idx(1 - par)
                issue_gathers(1 - par)

            wait_gathers(par)

            @pl.when(c + 2 < NCHUNK)
            def _():
                issue_idx(c + 2, par)

            @pl.when(c >= 2)
            def _():
                wait_out(par)

            compute(par)
            issue_out(c, par)

    wait_out(0)
    wait_out(1)


def _gather_sum(ids2d, table_n, gamma, beta):
    mesh = plsc.VectorSubcoreMesh(core_axis_name="c", subcore_axis_name="s")
    f = functools.partial(
        pl.kernel,
        out_type=jax.ShapeDtypeStruct((S, D), jnp.float32),
        mesh=mesh,
        compiler_params=_SC_PARAMS,
        scratch_types=[
            pltpu.VMEM((2, GPC, TPG), jnp.int32),
            pltpu.VMEM((2, CH * L, W), jnp.int32),
            pltpu.VMEM((2, CH, D), jnp.float32),
            pltpu.VMEM((2 * D,), jnp.float32),
            [pltpu.SemaphoreType.DMA, pltpu.SemaphoreType.DMA],
            [pltpu.SemaphoreType.DMA, pltpu.SemaphoreType.DMA],
            [pltpu.SemaphoreType.DMA, pltpu.SemaphoreType.DMA],
        ],
    )(_seg_sum_body)
    return f(ids2d, table_n, gamma, beta)


def kernel(input_ids, table, gamma, beta):
    table_n = _normalize_table(table)
    ids2d = input_ids.reshape(S * L // TPG, TPG)
    out = _gather_sum(ids2d, table_n, gamma, beta)
    return out.reshape(B, F, D)


# speedup vs baseline: 1.0090x; 1.0090x over previous
import functools

import jax
import jax.numpy as jnp
from jax import lax
from jax.experimental import pallas as pl
from jax.experimental.pallas import tpu as pltpu
from jax.experimental.pallas import tpu_sc as plsc

VOCAB = 100000
D = 64
W = D // 2
LN_EPS = 1e-12

B, F, L = 1024, 26, 50
S = B * F

NC, NS = 2, 16
NW = NC * NS

ROWS_MAIN = 3200
ROWS_LAST = VOCAB - 31 * ROWS_MAIN
RC = 160

SEG_W = S // NW
CH = 16
NCHUNK = SEG_W // CH
TPG = 100
GPC = CH * L // TPG

_SC_PARAMS = pltpu.CompilerParams(
    use_tc_tiling_on_sc=False, needs_layout_passes=False)


def _ln_body(tab_hbm, out_hbm, in_v, out_v, isem, osem):
    wid = lax.axis_index("s") * NC + lax.axis_index("c")
    base = wid * ROWS_MAIN
    nch = jnp.where(wid < NW - 1, ROWS_MAIN // RC, ROWS_LAST // RC)

    iota = lax.iota(jnp.int32, 16)
    half = jnp.full((16,), 0.5, jnp.float32)
    three_half = jnp.full((16,), 1.5, jnp.float32)
    inv_d = jnp.full((16,), 1.0 / D, jnp.float32)
    eps = jnp.full((16,), LN_EPS, jnp.float32)
    zero = jnp.zeros((16,), jnp.float32)

    def issue_in(c, p):
        row0 = pl.multiple_of(base + c * RC, 8)
        return pltpu.async_copy(
            tab_hbm.at[pl.ds(row0, RC)], in_v.at[p, :, pl.ds(0, D)], isem[p])

    def wait_in(p):
        pltpu.make_async_copy(
            tab_hbm.at[pl.ds(0, RC)], in_v.at[p, :, pl.ds(0, D)],
            isem[p]).wait()

    def issue_out(c, p):
        row0 = pl.multiple_of(base + c * RC, 8)
        return pltpu.async_copy(
            out_v.at[p, :, pl.ds(0, W)], out_hbm.at[pl.ds(row0, RC)], osem[p])

    def wait_out(p):
        pltpu.make_async_copy(
            out_v.at[p, :, pl.ds(0, W)], out_hbm.at[pl.ds(0, RC)],
            osem[p]).wait()

    def ln_group(p, r0):
        rowv = iota + r0
        ss = [zero] * 8
        qq = [zero] * 8
        for j in range(16):
            jm = (iota + j) & 15
            for q4 in range(4):
                x = plsc.load_gather(in_v.at[p], [rowv, jm + 16 * q4])
                k = (4 * j + q4) % 8
                ss[k] = ss[k] + x
                qq[k] = x * x + qq[k]
        while len(ss) > 1:
            ss = [a + b for a, b in zip(ss[::2], ss[1::2])]
            qq = [a + b for a, b in zip(qq[::2], qq[1::2])]
        mu = ss[0] * inv_d
        var = qq[0] * inv_d - mu * mu
        t = jnp.maximum(var, 0.0) + eps
        y = plsc.bitcast(
            jnp.full((16,), 0x5F3759DF, jnp.int32)
            - (plsc.bitcast(t, jnp.int32) >> 1), jnp.float32)
        ht = t * half
        for _ in range(2):
            y = y * (three_half - ht * y * y)
        c0 = mu * y

        for j in range(16):
            jm = (iota + j) & 15
            for hb in range(2):
                xa = plsc.load_gather(in_v.at[p], [rowv, jm + 32 * hb])
                xb = plsc.load_gather(in_v.at[p], [rowv, jm + (32 * hb + 16)])
                ya = xa * y - c0
                yb = xb * y - c0
                wrd = plsc.bitcast(
                    plsc.pack(ya, yb, format=plsc.PackFormat.INTERLEAVED),
                    jnp.int32)
                plsc.store_scatter(out_v.at[p], [rowv, jm + 16 * hb], wrd)

    issue_in(0, 0)
    issue_in(1, 1)

    @pl.loop(0, NCHUNK_LN_MAX, step=2)
    def _chunk(c0):
        for par in (0, 1):
            c = c0 + par

            @pl.when(c < nch)
            def _():
                wait_in(par)

                @pl.when(c >= 2)
                def _():
                    wait_out(par)

                @pl.loop(0, RC // 16)
                def _grp(g):
                    ln_group(par, g * 16)

                issue_out(c, par)

                @pl.when(c + 2 < nch)
                def _():
                    issue_in(c + 2, par)

    wait_out(0)
    wait_out(1)


NCHUNK_LN_MAX = ROWS_MAIN // RC


def _normalize_table(table):
    mesh = plsc.VectorSubcoreMesh(core_axis_name="c", subcore_axis_name="s")
    f = functools.partial(
        pl.kernel,
        out_type=jax.ShapeDtypeStruct((VOCAB, W), jnp.int32),
        mesh=mesh,
        compiler_params=_SC_PARAMS,
        scratch_types=[
            pltpu.VMEM((2, RC, D + 1), jnp.float32),
            pltpu.VMEM((2, RC, W + 1), jnp.int32),
            [pltpu.SemaphoreType.DMA, pltpu.SemaphoreType.DMA],
            [pltpu.SemaphoreType.DMA, pltpu.SemaphoreType.DMA],
        ],
    )(_ln_body)
    return f(table)


def _seg_sum_body(ids_hbm, tabn_hbm, gamma_hbm, beta_hbm, out_hbm,
                  idx_v, rows_v, out_v, gb_v, isem, gsem, osem):
    wid = lax.axis_index("s") * NC + lax.axis_index("c")
    base_seg = wid * SEG_W
    base_row = wid * (SEG_W * L // TPG)

    pltpu.sync_copy(gamma_hbm, gb_v.at[pl.ds(0, D)])
    pltpu.sync_copy(beta_hbm, gb_v.at[pl.ds(D, D)])
    gvec = [gb_v[pl.ds(16 * k, 16)] for k in range(4)]
    bvec = [gb_v[pl.ds(D + 16 * k, 16)] * 50.0 for k in range(4)]

    def issue_idx(c, p):
        row0 = pl.multiple_of(base_row + c * GPC, 8)
        return pltpu.async_copy(
            ids_hbm.at[pl.ds(row0, GPC)], idx_v.at[p], isem[p])

    def wait_idx(p):
        pltpu.make_async_copy(
            ids_hbm.at[pl.ds(0, GPC)], idx_v.at[p], isem[p]).wait()

    def issue_gathers(p):
        for j in range(GPC):
            pltpu.async_copy(
                tabn_hbm.at[idx_v.at[p, j]],
                rows_v.at[p, pl.ds(j * TPG, TPG)],
                gsem[p])

    def wait_gathers(p):
        for j in range(GPC):
            pltpu.make_async_copy(
                tabn_hbm.at[idx_v.at[p, j]],
                rows_v.at[p, pl.ds(j * TPG, TPG)],
                gsem[p]).wait()

    def issue_out(c, p):
        seg0 = pl.multiple_of(base_seg + c * CH, 8)
        return pltpu.async_copy(
            out_v.at[p], out_hbm.at[pl.ds(seg0, CH)], osem[p])

    def wait_out(p):
        pltpu.make_async_copy(
            out_v.at[p], out_hbm.at[pl.ds(0, CH)], osem[p]).wait()

    def compute(p):
        @pl.loop(0, CH)
        def _seg(s):
            base = s * L
            acc = [jnp.zeros((16,), jnp.float32) for _ in range(4)]
            for t in range(L // 2):
                r0 = base + 2 * t
                for h in range(2):
                    a = plsc.bitcast(
                        rows_v[p, r0, pl.ds(h * 16, 16)], jnp.bfloat16)
                    b = plsc.bitcast(
                        rows_v[p, r0 + 1, pl.ds(h * 16, 16)], jnp.bfloat16)
                    pair = plsc.bitcast(a + b, jnp.int32)
                    lo = plsc.bitcast(pair << 16, jnp.float32)
                    hi = plsc.bitcast(pair & jnp.int32(-65536), jnp.float32)
                    acc[2 * h] = acc[2 * h] + lo
                    acc[2 * h + 1] = acc[2 * h + 1] + hi
            for k in range(4):
                out_v[p, s, pl.ds(16 * k, 16)] = acc[k] * gvec[k] + bvec[k]

    issue_idx(0, 0)
    wait_idx(0)
    issue_gathers(0)
    issue_idx(1, 1)

    @pl.loop(0, NCHUNK, step=2)
    def _chunk(c0):
        for par in (0, 1):
            c = c0 + par

            @pl.when(c + 1 < NCHUNK)
            def _():
                wait_idx(1 - par)
                issue_gathers(1 - par)

            wait_gathers(par)

            @pl.when(c + 2 < NCHUNK)
            def _():
                issue_idx(c + 2, par)

            @pl.when(c >= 2)
            def _():
                wait_out(par)

            compute(par)
            issue_out(c, par)

    wait_out(0)
    wait_out(1)


def _gather_sum(ids2d, table_n, gamma, beta):
    mesh = plsc.VectorSubcoreMesh(core_axis_name="c", subcore_axis_name="s")
    f = functools.partial(
        pl.kernel,
        out_type=jax.ShapeDtypeStruct((S, D), jnp.float32),
        mesh=mesh,
        compiler_params=_SC_PARAMS,
        scratch_types=[
            pltpu.VMEM((2, GPC, TPG), jnp.int32),
            pltpu.VMEM((2, CH * L, W), jnp.int32),
            pltpu.VMEM((2, CH, D), jnp.float32),
            pltpu.VMEM((2 * D,), jnp.float32),
            [pltpu.SemaphoreType.DMA, pltpu.SemaphoreType.DMA],
            [pltpu.SemaphoreType.DMA, pltpu.SemaphoreType.DMA],
            [pltpu.SemaphoreType.DMA, pltpu.SemaphoreType.DMA],
        ],
    )(_seg_sum_body)
    return f(ids2d, table_n, gamma, beta)


def kernel(input_ids, table, gamma, beta):
    table_n = _normalize_table(table)
    ids2d = input_ids.reshape(S * L // TPG, TPG)
    out = _gather_sum(ids2d, table_n, gamma, beta)
    return out.reshape(B, F, D)


# speedup vs baseline: 1.0641x; 1.0546x over previous
import functools

import jax
import jax.numpy as jnp
from jax import lax
from jax.experimental import pallas as pl
from jax.experimental.pallas import tpu as pltpu
from jax.experimental.pallas import tpu_sc as plsc

VOCAB = 100000
D = 64
W = D // 2
LN_EPS = 1e-12

B, F, L = 1024, 26, 50
S = B * F

NC, NS = 2, 16
NW = NC * NS

ROWS_MAIN = 3200
ROWS_LAST = VOCAB - 31 * ROWS_MAIN
RC = 160

SEG_W = S // NW
CH = 32
NCHUNK = SEG_W // CH
TPG = 100
GPC = CH * L // TPG

_SC_PARAMS = pltpu.CompilerParams(
    use_tc_tiling_on_sc=False, needs_layout_passes=False)


def _ln_body(tab_hbm, out_hbm, in_v, out_v, isem, osem):
    wid = lax.axis_index("s") * NC + lax.axis_index("c")
    base = wid * ROWS_MAIN
    nch = jnp.where(wid < NW - 1, ROWS_MAIN // RC, ROWS_LAST // RC)

    iota = lax.iota(jnp.int32, 16)
    half = jnp.full((16,), 0.5, jnp.float32)
    three_half = jnp.full((16,), 1.5, jnp.float32)
    inv_d = jnp.full((16,), 1.0 / D, jnp.float32)
    eps = jnp.full((16,), LN_EPS, jnp.float32)
    zero = jnp.zeros((16,), jnp.float32)

    def issue_in(c, p):
        row0 = pl.multiple_of(base + c * RC, 8)
        return pltpu.async_copy(
            tab_hbm.at[pl.ds(row0, RC)], in_v.at[p, :, pl.ds(0, D)], isem[p])

    def wait_in(p):
        pltpu.make_async_copy(
            tab_hbm.at[pl.ds(0, RC)], in_v.at[p, :, pl.ds(0, D)],
            isem[p]).wait()

    def issue_out(c, p):
        row0 = pl.multiple_of(base + c * RC, 8)
        return pltpu.async_copy(
            out_v.at[p, :, pl.ds(0, W)], out_hbm.at[pl.ds(row0, RC)], osem[p])

    def wait_out(p):
        pltpu.make_async_copy(
            out_v.at[p, :, pl.ds(0, W)], out_hbm.at[pl.ds(0, RC)],
            osem[p]).wait()

    def ln_group(p, r0):
        rowv = iota + r0
        ss = [zero] * 8
        qq = [zero] * 8
        for j in range(16):
            jm = (iota + j) & 15
            for q4 in range(4):
                x = plsc.load_gather(in_v.at[p], [rowv, jm + 16 * q4])
                k = (4 * j + q4) % 8
                ss[k] = ss[k] + x
                qq[k] = x * x + qq[k]
        while len(ss) > 1:
            ss = [a + b for a, b in zip(ss[::2], ss[1::2])]
            qq = [a + b for a, b in zip(qq[::2], qq[1::2])]
        mu = ss[0] * inv_d
        var = qq[0] * inv_d - mu * mu
        t = jnp.maximum(var, 0.0) + eps
        y = plsc.bitcast(
            jnp.full((16,), 0x5F3759DF, jnp.int32)
            - (plsc.bitcast(t, jnp.int32) >> 1), jnp.float32)
        ht = t * half
        for _ in range(2):
            y = y * (three_half - ht * y * y)
        c0 = mu * y

        for j in range(16):
            jm = (iota + j) & 15
            for hb in range(2):
                xa = plsc.load_gather(in_v.at[p], [rowv, jm + 32 * hb])
                xb = plsc.load_gather(in_v.at[p], [rowv, jm + (32 * hb + 16)])
                ya = xa * y - c0
                yb = xb * y - c0
                wrd = plsc.bitcast(
                    plsc.pack(ya, yb, format=plsc.PackFormat.INTERLEAVED),
                    jnp.int32)
                plsc.store_scatter(out_v.at[p], [rowv, jm + 16 * hb], wrd)

    issue_in(0, 0)
    issue_in(1, 1)

    @pl.loop(0, NCHUNK_LN_MAX, step=2)
    def _chunk(c0):
        for par in (0, 1):
            c = c0 + par

            @pl.when(c < nch)
            def _():
                wait_in(par)

                @pl.when(c >= 2)
                def _():
                    wait_out(par)

                @pl.loop(0, RC // 16)
                def _grp(g):
                    ln_group(par, g * 16)

                issue_out(c, par)

                @pl.when(c + 2 < nch)
                def _():
                    issue_in(c + 2, par)

    wait_out(0)
    wait_out(1)


NCHUNK_LN_MAX = ROWS_MAIN // RC


def _normalize_table(table):
    mesh = plsc.VectorSubcoreMesh(core_axis_name="c", subcore_axis_name="s")
    f = functools.partial(
        pl.kernel,
        out_type=jax.ShapeDtypeStruct((VOCAB, W), jnp.int32),
        mesh=mesh,
        compiler_params=_SC_PARAMS,
        scratch_types=[
            pltpu.VMEM((2, RC, D + 1), jnp.float32),
            pltpu.VMEM((2, RC, W + 1), jnp.int32),
            [pltpu.SemaphoreType.DMA, pltpu.SemaphoreType.DMA],
            [pltpu.SemaphoreType.DMA, pltpu.SemaphoreType.DMA],
        ],
    )(_ln_body)
    return f(table)


def _seg_sum_body(ids_hbm, tabn_hbm, gamma_hbm, beta_hbm, out_hbm,
                  idx_v, rows_v, out_v, gb_v, isem, gsem, osem):
    wid = lax.axis_index("s") * NC + lax.axis_index("c")
    base_seg = wid * SEG_W
    base_row = wid * (SEG_W * L // TPG)

    pltpu.sync_copy(gamma_hbm, gb_v.at[pl.ds(0, D)])
    pltpu.sync_copy(beta_hbm, gb_v.at[pl.ds(D, D)])
    gvec = [gb_v[pl.ds(16 * k, 16)] for k in range(4)]
    bvec = [gb_v[pl.ds(D + 16 * k, 16)] * 50.0 for k in range(4)]

    def issue_idx(c, p):
        row0 = pl.multiple_of(base_row + c * GPC, 8)
        return pltpu.async_copy(
            ids_hbm.at[pl.ds(row0, GPC)], idx_v.at[p], isem[p])

    def wait_idx(p):
        pltpu.make_async_copy(
            ids_hbm.at[pl.ds(0, GPC)], idx_v.at[p], isem[p]).wait()

    def issue_gathers(p):
        for j in range(GPC):
            pltpu.async_copy(
                tabn_hbm.at[idx_v.at[p, j]],
                rows_v.at[p, pl.ds(j * TPG, TPG)],
                gsem[p])

    def wait_gathers(p):
        for j in range(GPC):
            pltpu.make_async_copy(
                tabn_hbm.at[idx_v.at[p, j]],
                rows_v.at[p, pl.ds(j * TPG, TPG)],
                gsem[p]).wait()

    def issue_out(c, p):
        row0 = pl.multiple_of((base_seg + c * CH) // 2, 8)
        return pltpu.async_copy(
            out_v.at[p], out_hbm.at[pl.ds(row0, CH // 2)], osem[p])

    def wait_out(p):
        pltpu.make_async_copy(
            out_v.at[p], out_hbm.at[pl.ds(0, CH // 2)], osem[p]).wait()

    def compute(p):
        @pl.loop(0, CH)
        def _seg(s):
            base = s * L
            acc = [jnp.zeros((16,), jnp.float32) for _ in range(4)]
            for t in range(L // 2):
                r0 = base + 2 * t
                for h in range(2):
                    a = plsc.bitcast(
                        rows_v[p, r0, pl.ds(h * 16, 16)], jnp.bfloat16)
                    b = plsc.bitcast(
                        rows_v[p, r0 + 1, pl.ds(h * 16, 16)], jnp.bfloat16)
                    pair = plsc.bitcast(a + b, jnp.int32)
                    lo = plsc.bitcast(pair << 16, jnp.float32)
                    hi = plsc.bitcast(pair & jnp.int32(-65536), jnp.float32)
                    acc[2 * h] = acc[2 * h] + lo
                    acc[2 * h + 1] = acc[2 * h + 1] + hi
            srow = s >> 1
            soff = (s & 1) * D
            for k in range(4):
                out_v[p, srow, pl.ds(soff + 16 * k, 16)] = (
                    acc[k] * gvec[k] + bvec[k])

    issue_idx(0, 0)
    wait_idx(0)
    issue_gathers(0)
    issue_idx(1, 1)

    @pl.loop(0, NCHUNK, step=2)
    def _chunk(c0):
        for par in (0, 1):
            c = c0 + par

            @pl.when(c + 1 < NCHUNK)
            def _():
                wait_idx(1 - par)
                issue_gathers(1 - par)

            wait_gathers(par)

            @pl.when(c + 2 < NCHUNK)
            def _():
                issue_idx(c + 2, par)

            @pl.when(c >= 2)
            def _():
                wait_out(par)

            compute(par)
            issue_out(c, par)

    wait_out(0)
    wait_out(1)


def _gather_sum(ids2d, table_n, gamma, beta):
    mesh = plsc.VectorSubcoreMesh(core_axis_name="c", subcore_axis_name="s")
    f = functools.partial(
        pl.kernel,
        out_type=jax.ShapeDtypeStruct((S // 2, 2 * D), jnp.float32),
        mesh=mesh,
        compiler_params=_SC_PARAMS,
        scratch_types=[
            pltpu.VMEM((2, GPC, TPG), jnp.int32),
            pltpu.VMEM((2, CH * L, W), jnp.int32),
            pltpu.VMEM((2, CH // 2, 2 * D), jnp.float32),
            pltpu.VMEM((2 * D,), jnp.float32),
            [pltpu.SemaphoreType.DMA, pltpu.SemaphoreType.DMA],
            [pltpu.SemaphoreType.DMA, pltpu.SemaphoreType.DMA],
            [pltpu.SemaphoreType.DMA, pltpu.SemaphoreType.DMA],
        ],
    )(_seg_sum_body)
    return f(ids2d, table_n, gamma, beta)


def kernel(input_ids, table, gamma, beta):
    table_n = _normalize_table(table)
    ids2d = input_ids.reshape(S * L // TPG, TPG)
    out = _gather_sum(ids2d, table_n, gamma, beta)
    return out.reshape(B, F, D)
